# baseline (speedup 1.0000x reference)
import jax
import jax.numpy as jnp
from jax.experimental import pallas as pl
from jax.experimental.pallas import tpu as pltpu

_W = 958
_PW = 925


def _net_kernel(x_ref, cw_ref, cb_ref, fw_ref, fb_ref, out_ref):
    x = x_ref[...]

    slabs = [x[:, 32 * ki + kj:32 * ki + kj + _W]
             for ki in range(3) for kj in range(3)]

    acc = None
    for c in range(3):
        z = None
        for t in range(9):
            prod = slabs[t] * cw_ref[c * 9 + t]
            z = prod if z is None else z + prod
        m = jnp.maximum(z[:, :_PW + 1], z[:, 32:32 + _PW + 1])
        p = jnp.maximum(m[:, :_PW], m[:, 1:_PW + 1])
        r = jnp.maximum(p + cb_ref[c], 0.0)
        d = jnp.dot(r, fw_ref[c], preferred_element_type=jnp.float32)
        acc = d if acc is None else acc + d
    out_ref[...] = acc + fb_ref[...]


def kernel(x, conv_w, conv_b, fc_w, fc_b):
    N = x.shape[0]
    xf = x.reshape(N, 1024).astype(jnp.float32)

    BM = 512
    n_pad = pl.cdiv(N, BM) * BM
    if n_pad != N:
        xf = jnp.pad(xf, ((0, n_pad - N), (0, 0)))

    cw = conv_w.reshape(27).astype(jnp.float32)
    cb = conv_b.reshape(3).astype(jnp.float32)

    t = fc_w.reshape(10, 3, 15, 15).astype(jnp.float32)
    tt = t.transpose(1, 2, 3, 0).reshape(3, 225, 10)
    ij = jnp.arange(15)
    q = (64 * ij[:, None] + 2 * ij[None, :]).reshape(-1)
    fw = jnp.zeros((3, _PW, 16), jnp.float32).at[:, q, :10].set(tt)
    fb = jnp.zeros((1, 16), jnp.float32).at[0, :10].set(fc_b.astype(jnp.float32))

    out = pl.pallas_call(
        _net_kernel,
        out_shape=jax.ShapeDtypeStruct((n_pad, 16), jnp.float32),
        grid=(n_pad // BM,),
        in_specs=[
            pl.BlockSpec((BM, 1024), lambda n: (n, 0)),
            pl.BlockSpec(memory_space=pltpu.MemorySpace.SMEM),
            pl.BlockSpec(memory_space=pltpu.MemorySpace.SMEM),
            pl.BlockSpec((3, _PW, 16), lambda n: (0, 0, 0)),
            pl.BlockSpec((1, 16), lambda n: (0, 0)),
        ],
        out_specs=pl.BlockSpec((BM, 16), lambda n: (n, 0)),
        compiler_params=pltpu.CompilerParams(
            dimension_semantics=("parallel",),
            vmem_limit_bytes=64 * 1024 * 1024),
    )(xf, cw, cb, fw, fb)

    return out[:N, :10]
